# Initial kernel scaffold; baseline (speedup 1.0000x reference)
import jax
import jax.numpy as jnp
from jax.experimental import pallas as pl


def kernel(x, edge_index, W1, b1, W2, b2):
    raise NotImplementedError("write your pallas kernel here")


# speedup vs baseline: 3.5327x; 3.5327x over previous
import functools

import jax
import jax.numpy as jnp
from jax import lax
from jax.experimental import pallas as pl
from jax.experimental.pallas import tpu as pltpu
from jax.experimental.pallas import tpu_sc as plsc

N = 10000
D = 256
H = 128
E = 160000
NSUB = 16
B = 64
E_PAD = 163840
CHUNKS = E_PAD // B
CHUNKS_PER_SUB = CHUNKS // NSUB
ACC_ROWS = 10240
ROWS_PER_SUB = ACC_ROWS // NSUB
PHASES = 5
CPB = CHUNKS_PER_SUB // PHASES

_mesh = plsc.VectorSubcoreMesh(core_axis_name="c", subcore_axis_name="s")


@functools.partial(
    pl.kernel,
    out_type=jax.ShapeDtypeStruct((2, ACC_ROWS, H), jnp.float32),
    mesh=_mesh,
    scratch_types=[
        pltpu.VMEM((CPB, B), jnp.int32),
        pltpu.VMEM((CPB, B), jnp.int32),
        pltpu.VMEM((4, B, H), jnp.float32),
        pltpu.VMEM_SHARED((ACC_ROWS, H), jnp.float32),
        pltpu.SemaphoreType.DMA,
        pltpu.SemaphoreType.DMA,
        pltpu.SemaphoreType.DMA,
        pltpu.SemaphoreType.DMA,
        pltpu.SemaphoreType.DMA,
        pltpu.SemaphoreType.DMA,
        pltpu.SemaphoreType.DMA,
        pltpu.SemaphoreType.DMA,
    ],
)
def _propagate_sc(xa_hbm, xb_hbm, src_hbm, dst_hbm, out_hbm,
                  sidx, didx, rows, acc,
                  sg0, sg1, sg2, sg3, ss0, ss1, ss2, ss3):
    sg = [sg0, sg1, sg2, sg3]
    ss = [ss0, ss1, ss2, ss3]
    cid = lax.axis_index("c")
    sid = lax.axis_index("s")

    zvec = jnp.zeros((16,), jnp.float32)

    @pl.loop(0, B)
    def _(r):
        @pl.loop(0, H // 16)
        def _(c):
            rows[0, r, pl.ds(c * 16, 16)] = zvec

    @pl.loop(0, ROWS_PER_SUB // B)
    def _(i):
        pltpu.sync_copy(rows.at[0],
                        acc.at[pl.ds(sid * ROWS_PER_SUB + i * B, B)])


    plsc.subcore_barrier()

    def edge_loop(x_hbm):
        @pl.loop(0, PHASES)
        def _(ph):
            base = sid * CHUNKS_PER_SUB + ph * CPB
            pltpu.sync_copy(src_hbm.at[pl.ds(base, CPB)], sidx)
            pltpu.sync_copy(dst_hbm.at[pl.ds(base, CPB)], didx)
            pltpu.async_copy(x_hbm.at[sidx.at[0]], rows.at[0], sg[0])
            pltpu.async_copy(x_hbm.at[sidx.at[1]], rows.at[1], sg[1])

            @pl.loop(0, CPB, step=4)
            def _(j):
                for s in range(4):
                    c = j + s
                    s2 = (s + 2) % 4
                    pltpu.make_async_copy(x_hbm.at[sidx.at[c]], rows.at[s],
                                          sg[s]).wait()
                    pltpu.async_copy(rows.at[s], acc.at[didx.at[c]], ss[s],
                                     add=True)

                    @pl.when(c + 2 < CPB)
                    def _(c=c, s2=s2):
                        @pl.when(c >= 2)
                        def _():
                            pltpu.make_async_copy(
                                rows.at[s2], acc.at[didx.at[c - 2]],
                                ss[s2]).wait()

                        pltpu.async_copy(x_hbm.at[sidx.at[c + 2]],
                                         rows.at[s2], sg[s2])

            for s in range(4):
                pltpu.make_async_copy(rows.at[s],
                                      acc.at[didx.at[CPB - 4 + s]],
                                      ss[s]).wait()

    @pl.when(cid == 0)
    def _():
        edge_loop(xa_hbm)

    @pl.when(cid == 1)
    def _():
        edge_loop(xb_hbm)

    plsc.subcore_barrier()

    pltpu.sync_copy(
        acc.at[pl.ds(sid * ROWS_PER_SUB, ROWS_PER_SUB)],
        out_hbm.at[cid].at[pl.ds(sid * ROWS_PER_SUB, ROWS_PER_SUB)],
    )


ROW_BLK1 = 1024
ROW_BLK2 = 1000


def _lin_relu_body(h_ref, w_ref, b_ref, o_ref):
    xblk = jnp.concatenate([h_ref[0], h_ref[1]], axis=1)
    y = jnp.dot(xblk, w_ref[...], preferred_element_type=jnp.float32)
    y = jnp.maximum(y + b_ref[...], 0.0)
    o_ref[0] = y[:, :H]
    o_ref[1] = y[:, H:]


def _lin_lsm_body(h_ref, w_ref, b_ref, o_ref):
    xblk = jnp.concatenate([h_ref[0], h_ref[1]], axis=1)
    y = jnp.dot(xblk, w_ref[...], preferred_element_type=jnp.float32)
    y = y + b_ref[...]
    m = jnp.max(y, axis=1, keepdims=True)
    s = y - m
    lse = jnp.log(jnp.sum(jnp.exp(s), axis=1, keepdims=True))
    o_ref[...] = s - lse


def _linear_relu_tc(h, wt, b):
    return pl.pallas_call(
        _lin_relu_body,
        grid=(ACC_ROWS // ROW_BLK1,),
        in_specs=[
            pl.BlockSpec((2, ROW_BLK1, H), lambda i: (0, i, 0)),
            pl.BlockSpec((D, D), lambda i: (0, 0)),
            pl.BlockSpec((1, D), lambda i: (0, 0)),
        ],
        out_specs=pl.BlockSpec((2, ROW_BLK1, H), lambda i: (0, i, 0)),
        out_shape=jax.ShapeDtypeStruct((2, ACC_ROWS, H), jnp.float32),
    )(h, wt, b)


def _linear_lsm_tc(h, wt, b):
    return pl.pallas_call(
        _lin_lsm_body,
        grid=(N // ROW_BLK2,),
        in_specs=[
            pl.BlockSpec((2, ROW_BLK2, H), lambda i: (0, i, 0)),
            pl.BlockSpec((D, D), lambda i: (0, 0)),
            pl.BlockSpec((1, D), lambda i: (0, 0)),
        ],
        out_specs=pl.BlockSpec((ROW_BLK2, D), lambda i: (i, 0)),
        out_shape=jax.ShapeDtypeStruct((N, D), jnp.float32),
    )(h, wt, b)


def kernel(x, edge_index, W1, b1, W2, b2):
    src = edge_index[0].astype(jnp.int32)
    dst = edge_index[1].astype(jnp.int32)
    src_p = jnp.concatenate([src, jnp.zeros((E_PAD - E,), jnp.int32)])
    dst_p = jnp.concatenate(
        [dst, jnp.full((E_PAD - E,), ACC_ROWS - 1, jnp.int32)])
    src3 = src_p.reshape(CHUNKS, B)
    dst3 = dst_p.reshape(CHUNKS, B)

    xs = x.reshape(N, 2, H).transpose(1, 0, 2)
    w1t = W1.T
    w2t = W2.T
    b1r = b1.reshape(1, D)
    b2r = b2.reshape(1, D)

    h = _propagate_sc(xs[0], xs[1], src3, dst3)
    h = _linear_relu_tc(h, w1t, b1r)
    h = _propagate_sc(h[0], h[1], src3, dst3)
    return _linear_lsm_tc(h, w2t, b2r)
